# baseline (speedup 1.0000x reference)
import functools

import jax
import jax.numpy as jnp
from jax import lax
from jax.experimental import pallas as pl
from jax.experimental.pallas import tpu as pltpu
from jax.experimental.pallas import tpu_sc as plsc

B, L, D, F = 16, 512, 256, 256
T = 2048
ROWS = B * T
TABLE_ROWS = B * L + 8
ZERO_ROW = B * L

NC, NS = 2, 16
NW = NC * NS
RPW = ROWS // NW
CH = 128
NCH = RPW // CH


def _idx_body(tgt_ref, mel_ref, out_ref):
    b = pl.program_id(0)
    tgt = tgt_ref[0]
    r = lax.broadcasted_iota(jnp.int32, (L, L), 0)
    c = lax.broadcasted_iota(jnp.int32, (L, L), 1)
    tri = (r <= c).astype(jnp.float32)
    cum = jnp.dot(tgt, tri, preferred_element_type=jnp.float32)
    total = cum[0, L - 1]
    mel_last = (mel_ref[0] - 1).astype(jnp.float32)
    t2 = lax.broadcasted_iota(jnp.int32, (T, L), 0).astype(jnp.float32)
    t2 = jnp.minimum(t2, mel_last)
    idx = jnp.sum((t2 >= cum).astype(jnp.float32), axis=1)
    t1 = jnp.minimum(
        lax.broadcasted_iota(jnp.int32, (1, T), 1).astype(jnp.float32),
        mel_last)
    mask = t1 < total
    gidx = jnp.where(mask, b * L + idx.astype(jnp.int32).reshape(1, T),
                     ZERO_ROW)
    out_ref[0] = gidx


def _compute_gather_idx(target_f32, mel):
    return pl.pallas_call(
        _idx_body,
        grid=(B,),
        in_specs=[
            pl.BlockSpec((1, 1, L), lambda b: (b, 0, 0)),
            pl.BlockSpec(memory_space=pltpu.SMEM),
        ],
        out_specs=pl.BlockSpec((1, 1, T), lambda b: (b, 0, 0)),
        out_shape=jax.ShapeDtypeStruct((B, 1, T), jnp.int32),
    )(target_f32, mel)


def _ln_relu(y, scale, bias):
    mu = jnp.mean(y, axis=-1, keepdims=True)
    d = y - mu
    var = jnp.mean(d * d, axis=-1, keepdims=True)
    return jnp.maximum(d * lax.rsqrt(var + 1e-5) * scale + bias, 0.0)


def _conv3(h, w, bias):
    z = jnp.zeros((1, h.shape[1]), jnp.float32)
    hprev = jnp.concatenate([z, h[:-1]], axis=0)
    hnext = jnp.concatenate([h[1:], z], axis=0)
    y = (jnp.dot(hprev, w[0], preferred_element_type=jnp.float32)
         + jnp.dot(h, w[1], preferred_element_type=jnp.float32)
         + jnp.dot(hnext, w[2], preferred_element_type=jnp.float32))
    return y + bias


def _pred_body(x_ref, w1_ref, b1_ref, s1_ref, g1_ref, w2_ref, b2_ref, s2_ref,
               g2_ref, lw_ref, lb_ref, out_ref):
    xb = x_ref[0]
    h = _ln_relu(_conv3(xb, w1_ref[...], b1_ref[...]), s1_ref[...], g1_ref[...])
    h = _ln_relu(_conv3(h, w2_ref[...], b2_ref[...]), s2_ref[...], g2_ref[...])
    dpo = jnp.maximum(jnp.sum(h * lw_ref[...], axis=-1) + lb_ref[0, 0], 0.0)
    out_ref[0] = dpo.reshape(1, L)


def _predictor(x, w1, b1, s1, g1, w2, b2, s2, g2, lw, lb):
    full = lambda a: pl.BlockSpec(a.shape, lambda b: (0,) * a.ndim)
    return pl.pallas_call(
        _pred_body,
        grid=(B,),
        in_specs=[pl.BlockSpec((1, L, D), lambda b: (b, 0, 0)),
                  full(w1), full(b1), full(s1), full(g1),
                  full(w2), full(b2), full(s2), full(g2),
                  full(lw), full(lb)],
        out_specs=pl.BlockSpec((1, 1, L), lambda b: (b, 0, 0)),
        out_shape=jax.ShapeDtypeStruct((B, 1, L), jnp.float32),
    )(x, w1, b1, s1, g1, w2, b2, s2, g2, lw, lb)


def _sc_gather_body(table_hbm, idx_hbm, out_hbm, idx_v, buf0, buf1, sem0, sem1):
    wid = lax.axis_index("s") * NC + lax.axis_index("c")
    base = wid * RPW
    pltpu.sync_copy(idx_hbm.at[pl.ds(base, RPW)], idx_v)
    bufs, sems = (buf0, buf1), (sem0, sem1)
    handles = [None, None]
    handles[0] = pltpu.async_copy(
        table_hbm.at[idx_v.at[pl.ds(0, CH)]], bufs[0], sems[0])
    for cnk in range(NCH):
        nxt = cnk + 1
        if nxt < NCH:
            handles[nxt % 2] = pltpu.async_copy(
                table_hbm.at[idx_v.at[pl.ds(nxt * CH, CH)]],
                bufs[nxt % 2], sems[nxt % 2])
        handles[cnk % 2].wait()
        pltpu.sync_copy(bufs[cnk % 2], out_hbm.at[pl.ds(base + cnk * CH, CH)])


@functools.cache
def _make_sc_gather():
    return pl.kernel(
        _sc_gather_body,
        mesh=plsc.VectorSubcoreMesh(core_axis_name="c", subcore_axis_name="s"),
        out_type=jax.ShapeDtypeStruct((ROWS, D), jnp.float32),
        scratch_types=[
            pltpu.VMEM((RPW,), jnp.int32),
            pltpu.VMEM((CH, D), jnp.float32),
            pltpu.VMEM((CH, D), jnp.float32),
            pltpu.SemaphoreType.DMA,
            pltpu.SemaphoreType.DMA,
        ],
    )


def _sc_gather(table, gidx):
    return _make_sc_gather()(table, gidx)


def kernel(x, alpha, target, mel_max_length, conv1_w, conv1_b, ln1_scale,
           ln1_bias, conv2_w, conv2_b, ln2_scale, ln2_bias, lin_w, lin_b):
    del alpha
    tgt_f = target.astype(jnp.float32).reshape(B, 1, L)
    mel = jnp.asarray(mel_max_length, jnp.int32).reshape(1)
    gidx = _compute_gather_idx(tgt_f, mel).reshape(ROWS)

    table = jnp.concatenate(
        [x.reshape(B * L, D), jnp.zeros((TABLE_ROWS - B * L, D), x.dtype)], 0)
    output = _sc_gather(table, gidx).reshape(B, T, D)

    w1 = jnp.transpose(conv1_w, (2, 1, 0))
    w2 = jnp.transpose(conv2_w, (2, 1, 0))
    dpo = _predictor(
        x, w1, conv1_b.reshape(1, F), ln1_scale.reshape(1, F),
        ln1_bias.reshape(1, F), w2, conv2_b.reshape(1, F),
        ln2_scale.reshape(1, F), ln2_bias.reshape(1, F),
        lin_w.reshape(1, F), lin_b.reshape(1, 1)).reshape(B, L)
    return output, dpo


# speedup vs baseline: 4.9864x; 4.9864x over previous
import functools

import jax
import jax.numpy as jnp
from jax import lax
from jax.experimental import pallas as pl
from jax.experimental.pallas import tpu as pltpu
from jax.experimental.pallas import tpu_sc as plsc

B, L, D, F = 16, 512, 256, 256
T = 2048
ROWS = B * T

NC, NS = 2, 16
NW = NC * NS
RPW = ROWS // NW
CH = 128
NCH = RPW // CH
NBUF = 3
VPC = CH // 16


def _ln_relu(y, scale, bias, ones_col):
    s1 = jnp.dot(y, ones_col, preferred_element_type=jnp.float32)[:, 0:1]
    s2 = jnp.dot(y * y, ones_col, preferred_element_type=jnp.float32)[:, 0:1]
    mu = s1 * (1.0 / F)
    var = s2 * (1.0 / F) - mu * mu
    return jnp.maximum((y - mu) * lax.rsqrt(var + 1e-5) * scale + bias, 0.0)


def _conv3(h, w, bias):
    z = jnp.zeros((1, h.shape[1]), h.dtype)
    hprev = jnp.concatenate([z, h[:-1]], axis=0)
    hnext = jnp.concatenate([h[1:], z], axis=0)
    y = (jnp.dot(hprev, w[0], preferred_element_type=jnp.float32)
         + jnp.dot(h, w[1], preferred_element_type=jnp.float32)
         + jnp.dot(hnext, w[2], preferred_element_type=jnp.float32))
    return y + bias


def _pred_body(x_ref, w1_ref, b1_ref, s1_ref, g1_ref, w2_ref, b2_ref, s2_ref,
               g2_ref, lw_ref, lb_ref, out_ref):
    ones_col = jnp.ones((F, 128), jnp.float32)
    xb = x_ref[0].astype(jnp.bfloat16)
    h = _ln_relu(_conv3(xb, w1_ref[...], b1_ref[...]), s1_ref[...],
                 g1_ref[...], ones_col)
    h = _ln_relu(_conv3(h.astype(jnp.bfloat16), w2_ref[...], b2_ref[...]),
                 s2_ref[...], g2_ref[...], ones_col)
    dpo = jnp.dot(h, lw_ref[...], preferred_element_type=jnp.float32)[:, 0:1]
    dpo = jnp.maximum(dpo + lb_ref[0, 0], 0.0)
    out_ref[0] = dpo.reshape(1, L)


def _predictor(x, w1, b1, s1, g1, w2, b2, s2, g2, lw, lb):
    full = lambda a: pl.BlockSpec(a.shape, lambda b: (0,) * a.ndim)
    return pl.pallas_call(
        _pred_body,
        grid=(B,),
        in_specs=[pl.BlockSpec((1, L, D), lambda b: (b, 0, 0)),
                  full(w1), full(b1), full(s1), full(g1),
                  full(w2), full(b2), full(s2), full(g2),
                  full(lw), full(lb)],
        out_specs=pl.BlockSpec((1, 1, L), lambda b: (b, 0, 0)),
        out_shape=jax.ShapeDtypeStruct((B, 1, L), jnp.float32),
    )(x, w1, b1, s1, g1, w2, b2, s2, g2, lw, lb)


def _sc_body(x_hbm, tgt_hbm, mel_hbm, out_hbm, tgt_v, cum_v, idx_v, mel_v,
             buf0, buf1, buf2, gs0, gs1, gs2, ss0, ss1, ss2):
    wid = lax.axis_index("s") * NC + lax.axis_index("c")
    b = wid // 2
    base_t = (wid % 2) * RPW
    out_base = wid * RPW
    bufs, gsems, ssems = (buf0, buf1, buf2), (gs0, gs1, gs2), (ss0, ss1, ss2)

    pltpu.sync_copy(tgt_hbm.at[pl.ds(b * L, L)], tgt_v)
    pltpu.sync_copy(mel_hbm, mel_v)
    mel_last = (mel_v[...][0] - 1).astype(jnp.float32)
    fRPW = jnp.float32(RPW)
    lane = lax.broadcasted_iota(jnp.int32, (16,), 0)

    def cs_body(i, carry):
        v = tgt_v[pl.ds(i * 16, 16)].astype(jnp.float32)
        cum_v[pl.ds(i * 16, 16)] = v
        for s in (1, 2, 4, 8):
            g = plsc.load_gather(cum_v, [jnp.maximum(lane - s, 0) + i * 16])
            v = v + jnp.where(lane >= s, g, 0.0)
            cum_v[pl.ds(i * 16, 16)] = v
        v = v + carry
        cum_v[pl.ds(i * 16, 16)] = v
        return v[15]

    total = lax.fori_loop(0, L // 16, cs_body, jnp.float32(0))
    base_f = (base_t * jnp.int32(1)).astype(jnp.float32)
    n_real = jnp.where(total > mel_last, fRPW,
                       jnp.clip(total - base_f, 0.0, fRPW)).astype(jnp.int32)

    def search_chunk(cnk):
        def bs(j, carry):
            te = jnp.minimum((base_t + j * 16 + lane).astype(jnp.float32),
                             mel_last)
            lo = jnp.zeros((16,), jnp.int32)
            for h in (256, 128, 64, 32, 16, 8, 4, 2, 1):
                cval = plsc.load_gather(cum_v, [lo + (h - 1)])
                lo = lo + jnp.where(cval <= te, h, 0)
            idx_v[pl.ds(j * 16, 16)] = jnp.minimum(lo, L - 1) + b * L
            return carry
        lax.fori_loop(cnk * VPC, (cnk + 1) * VPC, bs, 0)

    def gather(cnk, i):
        return pltpu.async_copy(
            x_hbm.at[idx_v.at[pl.ds(cnk * CH, CH)]], bufs[i], gsems[i])

    def store(cnk, i):
        return pltpu.async_copy(
            bufs[i], out_hbm.at[pl.ds(out_base + cnk * CH, CH)], ssems[i])

    def zero_tail(buf, zstart):
        def zrow(r, carry):
            for k in range(D // 16):
                buf[r, pl.ds(k * 16, 16)] = jnp.zeros((16,), jnp.float32)
            return carry
        lax.fori_loop(zstart, CH, zrow, 0)

    gh = [None] * NBUF
    sh = [None] * NBUF
    for cnk in range(NBUF):
        search_chunk(cnk)
        gh[cnk] = gather(cnk, cnk)
    for cnk in range(NCH):
        i = cnk % NBUF
        nxt = cnk + NBUF
        if nxt < NCH:
            search_chunk(nxt)
        gh[i].wait()
        zero_tail(bufs[i], jnp.clip(n_real - cnk * CH, 0, CH))
        sh[i] = store(cnk, i)
        if nxt < NCH:
            sh[i].wait()
            gh[i] = gather(nxt, i)
    for cnk in range(max(0, NCH - NBUF), NCH):
        sh[cnk % NBUF].wait()


@functools.cache
def _make_sc_expand():
    return pl.kernel(
        _sc_body,
        mesh=plsc.VectorSubcoreMesh(core_axis_name="c", subcore_axis_name="s"),
        compiler_params=pltpu.CompilerParams(needs_layout_passes=False),
        out_type=jax.ShapeDtypeStruct((ROWS, D), jnp.float32),
        scratch_types=[
            pltpu.VMEM((L,), jnp.int32),
            pltpu.VMEM((L,), jnp.float32),
            pltpu.VMEM((RPW,), jnp.int32),
            pltpu.VMEM((16,), jnp.int32),
            pltpu.VMEM((CH, D), jnp.float32),
            pltpu.VMEM((CH, D), jnp.float32),
            pltpu.VMEM((CH, D), jnp.float32),
            pltpu.SemaphoreType.DMA,
            pltpu.SemaphoreType.DMA,
            pltpu.SemaphoreType.DMA,
            pltpu.SemaphoreType.DMA,
            pltpu.SemaphoreType.DMA,
            pltpu.SemaphoreType.DMA,
        ],
    )


def _sc_expand(x_flat, tgt_flat, mel16):
    return _make_sc_expand()(x_flat, tgt_flat, mel16)


def kernel(x, alpha, target, mel_max_length, conv1_w, conv1_b, ln1_scale,
           ln1_bias, conv2_w, conv2_b, ln2_scale, ln2_bias, lin_w, lin_b):
    del alpha
    mel16 = jnp.full((16,), mel_max_length, jnp.int32)
    output = _sc_expand(x.reshape(B * L, D), target.reshape(B * L),
                        mel16).reshape(B, T, D)

    w1 = jnp.transpose(conv1_w, (2, 1, 0)).astype(jnp.bfloat16)
    w2 = jnp.transpose(conv2_w, (2, 1, 0)).astype(jnp.bfloat16)
    lw = jnp.pad(lin_w, ((0, 0), (0, 127)))
    dpo = _predictor(
        x, w1, conv1_b.reshape(1, F), ln1_scale.reshape(1, F),
        ln1_bias.reshape(1, F), w2, conv2_b.reshape(1, F),
        ln2_scale.reshape(1, F), ln2_bias.reshape(1, F),
        lw, lin_b.reshape(1, 1)).reshape(B, L)
    return output, dpo


# speedup vs baseline: 10.4717x; 2.1001x over previous
import functools

import jax
import jax.numpy as jnp
from jax import lax
from jax.experimental import pallas as pl
from jax.experimental.pallas import tpu as pltpu
from jax.experimental.pallas import tpu_sc as plsc

B, L, D, F = 16, 512, 256, 256
T = 2048
ROWS = B * T

NC, NS = 2, 16
NW = NC * NS
RPW = ROWS // NW
CH = 128
NCH = RPW // CH
NBUF = 2
VPC = CH // 16


def _ln_relu(y, scale, bias, ones_col):
    s1 = jnp.dot(y, ones_col, preferred_element_type=jnp.float32)[:, 0:1]
    s2 = jnp.dot(y * y, ones_col, preferred_element_type=jnp.float32)[:, 0:1]
    mu = s1 * (1.0 / F)
    var = s2 * (1.0 / F) - mu * mu
    return jnp.maximum((y - mu) * lax.rsqrt(var + 1e-5) * scale + bias, 0.0)


def _conv3(h, w, bias):
    z = jnp.zeros((1, h.shape[1]), h.dtype)
    hprev = jnp.concatenate([z, h[:-1]], axis=0)
    hnext = jnp.concatenate([h[1:], z], axis=0)
    y = (jnp.dot(hprev, w[0], preferred_element_type=jnp.float32)
         + jnp.dot(h, w[1], preferred_element_type=jnp.float32)
         + jnp.dot(hnext, w[2], preferred_element_type=jnp.float32))
    return y + bias


def _pred_body(x_ref, w1_ref, b1_ref, s1_ref, g1_ref, w2_ref, b2_ref, s2_ref,
               g2_ref, lw_ref, lb_ref, out_ref):
    ones_col = jnp.ones((F, 128), jnp.float32)
    xb = x_ref[0].astype(jnp.bfloat16)
    h = _ln_relu(_conv3(xb, w1_ref[...], b1_ref[...]), s1_ref[...],
                 g1_ref[...], ones_col)
    h = _ln_relu(_conv3(h.astype(jnp.bfloat16), w2_ref[...], b2_ref[...]),
                 s2_ref[...], g2_ref[...], ones_col)
    dpo = jnp.dot(h, lw_ref[...], preferred_element_type=jnp.float32)[:, 0:1]
    dpo = jnp.maximum(dpo + lb_ref[0, 0], 0.0)
    out_ref[0] = dpo.reshape(1, L)


def _predictor(x, w1, b1, s1, g1, w2, b2, s2, g2, lw, lb):
    full = lambda a: pl.BlockSpec(a.shape, lambda b: (0,) * a.ndim)
    return pl.pallas_call(
        _pred_body,
        grid=(B,),
        in_specs=[pl.BlockSpec((1, L, D), lambda b: (b, 0, 0)),
                  full(w1), full(b1), full(s1), full(g1),
                  full(w2), full(b2), full(s2), full(g2),
                  full(lw), full(lb)],
        out_specs=pl.BlockSpec((1, 1, L), lambda b: (b, 0, 0)),
        out_shape=jax.ShapeDtypeStruct((B, 1, L), jnp.float32),
    )(x, w1, b1, s1, g1, w2, b2, s2, g2, lw, lb)


def _sc_body(x_hbm, tgt_hbm, mel_hbm, out_hbm, tgt_v, cum_v, idx_v, mel_v,
             buf0, buf1, zbuf, gs0, gs1, ss0, ss1):
    wid = lax.axis_index("s") * NC + lax.axis_index("c")
    b = wid // 2
    half = wid % 2
    bufs, gsems, ssems = (buf0, buf1), (gs0, gs1), (ss0, ss1)

    def t_start(cnk):
        return (2 * cnk + half) * CH

    pltpu.sync_copy(tgt_hbm.at[pl.ds(b * L, L)], tgt_v)
    pltpu.sync_copy(mel_hbm, mel_v)
    mel_last = (mel_v[...][0] - 1).astype(jnp.float32)
    lane = lax.broadcasted_iota(jnp.int32, (16,), 0)

    def cs_body(i, carry):
        v = tgt_v[pl.ds(i * 16, 16)].astype(jnp.float32)
        cum_v[pl.ds(i * 16, 16)] = v
        for s in (1, 2, 4, 8):
            g = plsc.load_gather(cum_v, [jnp.maximum(lane - s, 0) + i * 16])
            v = v + jnp.where(lane >= s, g, 0.0)
            cum_v[pl.ds(i * 16, 16)] = v
        v = v + carry
        cum_v[pl.ds(i * 16, 16)] = v
        return v[15]

    total = lax.fori_loop(0, L // 16, cs_body, jnp.float32(0))

    def n_real(cnk):
        ts = jnp.float32(0) + t_start(cnk).astype(jnp.float32)
        return jnp.where(total > mel_last, jnp.float32(CH),
                         jnp.clip(total - ts, 0.0, jnp.float32(CH))
                         ).astype(jnp.int32)

    def search_chunk(cnk):
        def bs(j, carry):
            te = jnp.minimum(
                (t_start(cnk) + j * 16 + lane).astype(jnp.float32), mel_last)
            lo = jnp.zeros((16,), jnp.int32)
            for h in (256, 128, 64, 32, 16, 8, 4, 2, 1):
                cval = plsc.load_gather(cum_v, [lo + (h - 1)])
                lo = lo + jnp.where(cval <= te, h, 0)
            idx_v[pl.ds(cnk * CH + j * 16, 16)] = jnp.minimum(lo, L - 1) + b * L
            return carry
        lax.fori_loop(0, VPC, bs, 0)

    def gather(cnk, i):
        return pltpu.async_copy(
            x_hbm.at[idx_v.at[pl.ds(cnk * CH, CH)]], bufs[i], gsems[i])

    def out_slice(cnk):
        return out_hbm.at[pl.ds(b * T + t_start(cnk), CH)]

    def zero_tail(buf, zstart):
        def zrow(r, carry):
            for k in range(D // 16):
                buf[r, pl.ds(k * 16, 16)] = jnp.zeros((16,), jnp.float32)
            return carry
        lax.fori_loop(zstart, CH, zrow, 0)

    nr = [n_real(cnk) for cnk in range(NCH)]
    has_real = [nr[cnk] > 0 for cnk in range(NCH)]

    gh = [None] * NBUF
    for cnk in range(NBUF):
        @pl.when(has_real[cnk])
        def _(cnk=cnk):
            search_chunk(cnk)
            gather(cnk, cnk)
    zero_tail(zbuf, 0)

    for cnk in range(NCH):
        i = cnk % NBUF
        nxt = cnk + NBUF

        @pl.when(has_real[cnk])
        def _(cnk=cnk, i=i):
            pltpu.make_async_copy(
                x_hbm.at[idx_v.at[pl.ds(cnk * CH, CH)]], bufs[i],
                gsems[i]).wait()
            zero_tail(bufs[i], nr[cnk])
            pltpu.async_copy(bufs[i], out_slice(cnk), ssems[i])

        @pl.when(jnp.logical_not(has_real[cnk]))
        def _(cnk=cnk, i=i):
            pltpu.async_copy(zbuf, out_slice(cnk), ssems[i])

        if nxt < NCH:
            pltpu.make_async_copy(bufs[i], out_slice(cnk), ssems[i]).wait()

            @pl.when(has_real[nxt])
            def _(nxt=nxt, i=i):
                search_chunk(nxt)
                gather(nxt, i)
    for cnk in range(max(0, NCH - NBUF), NCH):
        pltpu.make_async_copy(
            bufs[cnk % NBUF], out_slice(cnk), ssems[cnk % NBUF]).wait()


@functools.cache
def _make_sc_expand():
    return pl.kernel(
        _sc_body,
        mesh=plsc.VectorSubcoreMesh(core_axis_name="c", subcore_axis_name="s"),
        compiler_params=pltpu.CompilerParams(needs_layout_passes=False,
                                             disable_bounds_checks=True),
        out_type=jax.ShapeDtypeStruct((ROWS, D), jnp.float32),
        scratch_types=[
            pltpu.VMEM((L,), jnp.int32),
            pltpu.VMEM((L,), jnp.float32),
            pltpu.VMEM((RPW,), jnp.int32),
            pltpu.VMEM((16,), jnp.int32),
            pltpu.VMEM((CH, D), jnp.float32),
            pltpu.VMEM((CH, D), jnp.float32),
            pltpu.VMEM((CH, D), jnp.float32),
            pltpu.SemaphoreType.DMA,
            pltpu.SemaphoreType.DMA,
            pltpu.SemaphoreType.DMA,
            pltpu.SemaphoreType.DMA,
        ],
    )


def _sc_expand(x_flat, tgt_flat, mel16):
    return _make_sc_expand()(x_flat, tgt_flat, mel16)


def kernel(x, alpha, target, mel_max_length, conv1_w, conv1_b, ln1_scale,
           ln1_bias, conv2_w, conv2_b, ln2_scale, ln2_bias, lin_w, lin_b):
    del alpha
    mel16 = jnp.full((16,), mel_max_length, jnp.int32)
    output = _sc_expand(x.reshape(B * L, D), target.reshape(B * L),
                        mel16).reshape(B, T, D)

    w1 = jnp.transpose(conv1_w, (2, 1, 0)).astype(jnp.bfloat16)
    w2 = jnp.transpose(conv2_w, (2, 1, 0)).astype(jnp.bfloat16)
    lw = jnp.pad(lin_w, ((0, 0), (0, 127)))
    dpo = _predictor(
        x, w1, conv1_b.reshape(1, F), ln1_scale.reshape(1, F),
        ln1_bias.reshape(1, F), w2, conv2_b.reshape(1, F),
        ln2_scale.reshape(1, F), ln2_bias.reshape(1, F),
        lw, lin_b.reshape(1, 1)).reshape(B, L)
    return output, dpo


# speedup vs baseline: 11.8401x; 1.1307x over previous
import functools

import jax
import jax.numpy as jnp
from jax import lax
from jax.experimental import pallas as pl
from jax.experimental.pallas import tpu as pltpu
from jax.experimental.pallas import tpu_sc as plsc

B, L, D, F = 16, 512, 256, 256
T = 2048
ROWS = B * T

NC, NS = 2, 16
NW = NC * NS
RPW = ROWS // NW
CH = 64
NCH = RPW // CH
NBUF = 4
PRE = 2
VPC = CH // 16


def _ln_relu(y, scale, bias, ones_col):
    s1 = jnp.dot(y, ones_col, preferred_element_type=jnp.float32)[:, 0:1]
    s2 = jnp.dot(y * y, ones_col, preferred_element_type=jnp.float32)[:, 0:1]
    mu = s1 * (1.0 / F)
    var = s2 * (1.0 / F) - mu * mu
    return jnp.maximum((y - mu) * lax.rsqrt(var + 1e-5) * scale + bias, 0.0)


def _conv3(h, w, bias):
    z = jnp.zeros((1, h.shape[1]), h.dtype)
    hprev = jnp.concatenate([z, h[:-1]], axis=0)
    hnext = jnp.concatenate([h[1:], z], axis=0)
    y = (jnp.dot(hprev, w[0], preferred_element_type=jnp.float32)
         + jnp.dot(h, w[1], preferred_element_type=jnp.float32)
         + jnp.dot(hnext, w[2], preferred_element_type=jnp.float32))
    return y + bias


PB = 2


def _pred_body(x_ref, w1_ref, b1_ref, s1_ref, g1_ref, w2_ref, b2_ref, s2_ref,
               g2_ref, lw_ref, lb_ref, out_ref):
    ones_col = jnp.ones((F, 128), jnp.float32)
    for sb in range(PB):
        xb = x_ref[sb].astype(jnp.bfloat16)
        h = _ln_relu(_conv3(xb, w1_ref[...], b1_ref[...]), s1_ref[...],
                     g1_ref[...], ones_col)
        h = _ln_relu(_conv3(h.astype(jnp.bfloat16), w2_ref[...], b2_ref[...]),
                     s2_ref[...], g2_ref[...], ones_col)
        dpo = jnp.dot(h, lw_ref[...],
                      preferred_element_type=jnp.float32)[:, 0:1]
        dpo = jnp.maximum(dpo + lb_ref[0, 0], 0.0)
        out_ref[sb] = dpo.reshape(1, L)


def _predictor(x, w1, b1, s1, g1, w2, b2, s2, g2, lw, lb):
    full = lambda a: pl.BlockSpec(a.shape, lambda b: (0,) * a.ndim)
    return pl.pallas_call(
        _pred_body,
        grid=(B // PB,),
        in_specs=[pl.BlockSpec((PB, L, D), lambda b: (b, 0, 0)),
                  full(w1), full(b1), full(s1), full(g1),
                  full(w2), full(b2), full(s2), full(g2),
                  full(lw), full(lb)],
        out_specs=pl.BlockSpec((PB, 1, L), lambda b: (b, 0, 0)),
        out_shape=jax.ShapeDtypeStruct((B, 1, L), jnp.float32),
    )(x, w1, b1, s1, g1, w2, b2, s2, g2, lw, lb)


def _sc_body(x_hbm, tgt_hbm, mel_hbm, out_hbm, tgt_v, cum_v, idx_v, mel_v,
             buf0, buf1, buf2, buf3, zbuf,
             gs0, gs1, gs2, gs3, ss0, ss1, ss2, ss3):
    wid = lax.axis_index("s") * NC + lax.axis_index("c")
    b = wid // 2
    half = wid % 2
    bufs = (buf0, buf1, buf2, buf3)
    gsems, ssems = (gs0, gs1, gs2, gs3), (ss0, ss1, ss2, ss3)

    def t_start(cnk):
        return (2 * cnk + half) * CH

    pltpu.sync_copy(tgt_hbm.at[pl.ds(b * L, L)], tgt_v)
    pltpu.sync_copy(mel_hbm, mel_v)
    mel_last = (mel_v[...][0] - 1).astype(jnp.float32)
    lane = lax.broadcasted_iota(jnp.int32, (16,), 0)

    def cs_body(i, carry):
        v = tgt_v[pl.ds(i * 16, 16)].astype(jnp.float32)
        cum_v[pl.ds(i * 16, 16)] = v
        for s in (1, 2, 4, 8):
            g = plsc.load_gather(cum_v, [jnp.maximum(lane - s, 0) + i * 16])
            v = v + jnp.where(lane >= s, g, 0.0)
            cum_v[pl.ds(i * 16, 16)] = v
        v = v + carry
        cum_v[pl.ds(i * 16, 16)] = v
        return v[15]

    total = lax.fori_loop(0, L // 16, cs_body, jnp.float32(0))

    def n_real(cnk):
        ts = jnp.float32(0) + t_start(cnk).astype(jnp.float32)
        return jnp.where(total > mel_last, jnp.float32(CH),
                         jnp.clip(total - ts, 0.0, jnp.float32(CH))
                         ).astype(jnp.int32)

    def search_chunk(cnk):
        def bs(j, carry):
            te = jnp.minimum(
                (t_start(cnk) + j * 16 + lane).astype(jnp.float32), mel_last)
            lo = jnp.zeros((16,), jnp.int32)
            for h in (256, 128, 64, 32, 16, 8, 4, 2, 1):
                cval = plsc.load_gather(cum_v, [lo + (h - 1)])
                lo = lo + jnp.where(cval <= te, h, 0)
            idx_v[pl.ds(cnk * CH + j * 16, 16)] = jnp.minimum(lo, L - 1) + b * L
            return carry
        lax.fori_loop(0, VPC, bs, 0)

    def gather(cnk, i):
        return pltpu.async_copy(
            x_hbm.at[idx_v.at[pl.ds(cnk * CH, CH)]], bufs[i], gsems[i])

    def out_slice(cnk):
        return out_hbm.at[pl.ds(b * T + t_start(cnk), CH)]

    def zero_tail(buf, zstart):
        def zrow(r, carry):
            for k in range(D // 16):
                buf[r, pl.ds(k * 16, 16)] = jnp.zeros((16,), jnp.float32)
            return carry
        lax.fori_loop(zstart, CH, zrow, 0)

    nr = [n_real(cnk) for cnk in range(NCH)]
    has_real = [nr[cnk] > 0 for cnk in range(NCH)]

    def wait_store(cnk, i):
        pltpu.make_async_copy(bufs[i], out_slice(cnk), ssems[i]).wait()

    for cnk in range(PRE):
        @pl.when(has_real[cnk])
        def _(cnk=cnk):
            search_chunk(cnk)
            gather(cnk, cnk % NBUF)
    zero_tail(zbuf, 0)

    store_waited = [False] * NCH
    for cnk in range(NCH):
        i = cnk % NBUF
        nxt = cnk + PRE
        if nxt < NCH:
            j = nxt % NBUF
            prev = nxt - NBUF
            if prev >= 0:
                wait_store(prev, j)
                store_waited[prev] = True

            @pl.when(has_real[nxt])
            def _(nxt=nxt, j=j):
                search_chunk(nxt)
                gather(nxt, j)

        @pl.when(has_real[cnk])
        def _(cnk=cnk, i=i):
            pltpu.make_async_copy(
                x_hbm.at[idx_v.at[pl.ds(cnk * CH, CH)]], bufs[i],
                gsems[i]).wait()
            zero_tail(bufs[i], nr[cnk])
            pltpu.async_copy(bufs[i], out_slice(cnk), ssems[i])

        @pl.when(jnp.logical_not(has_real[cnk]))
        def _(cnk=cnk, i=i):
            pltpu.async_copy(zbuf, out_slice(cnk), ssems[i])

    for cnk in range(NCH):
        if not store_waited[cnk]:
            wait_store(cnk, cnk % NBUF)


@functools.cache
def _make_sc_expand():
    return pl.kernel(
        _sc_body,
        mesh=plsc.VectorSubcoreMesh(core_axis_name="c", subcore_axis_name="s"),
        compiler_params=pltpu.CompilerParams(needs_layout_passes=False,
                                             disable_bounds_checks=True),
        out_type=jax.ShapeDtypeStruct((ROWS, D), jnp.float32),
        scratch_types=[
            pltpu.VMEM((L,), jnp.int32),
            pltpu.VMEM((L,), jnp.float32),
            pltpu.VMEM((RPW,), jnp.int32),
            pltpu.VMEM((16,), jnp.int32),
            pltpu.VMEM((CH, D), jnp.float32),
            pltpu.VMEM((CH, D), jnp.float32),
            pltpu.VMEM((CH, D), jnp.float32),
            pltpu.VMEM((CH, D), jnp.float32),
            pltpu.VMEM((CH, D), jnp.float32),
            pltpu.SemaphoreType.DMA,
            pltpu.SemaphoreType.DMA,
            pltpu.SemaphoreType.DMA,
            pltpu.SemaphoreType.DMA,
            pltpu.SemaphoreType.DMA,
            pltpu.SemaphoreType.DMA,
            pltpu.SemaphoreType.DMA,
            pltpu.SemaphoreType.DMA,
        ],
    )


def _sc_expand(x_flat, tgt_flat, mel16):
    return _make_sc_expand()(x_flat, tgt_flat, mel16)


def kernel(x, alpha, target, mel_max_length, conv1_w, conv1_b, ln1_scale,
           ln1_bias, conv2_w, conv2_b, ln2_scale, ln2_bias, lin_w, lin_b):
    del alpha
    mel16 = jnp.full((16,), mel_max_length, jnp.int32)
    output = _sc_expand(x.reshape(B * L, D), target.reshape(B * L),
                        mel16).reshape(B, T, D)

    w1 = jnp.transpose(conv1_w, (2, 1, 0)).astype(jnp.bfloat16)
    w2 = jnp.transpose(conv2_w, (2, 1, 0)).astype(jnp.bfloat16)
    lw = jnp.pad(lin_w, ((0, 0), (0, 127)))
    dpo = _predictor(
        x, w1, conv1_b.reshape(1, F), ln1_scale.reshape(1, F),
        ln1_bias.reshape(1, F), w2, conv2_b.reshape(1, F),
        ln2_scale.reshape(1, F), ln2_bias.reshape(1, F),
        lw, lin_b.reshape(1, 1)).reshape(B, L)
    return output, dpo


# speedup vs baseline: 11.9467x; 1.0090x over previous
import functools

import jax
import jax.numpy as jnp
from jax import lax
from jax.experimental import pallas as pl
from jax.experimental.pallas import tpu as pltpu
from jax.experimental.pallas import tpu_sc as plsc

B, L, D, F = 16, 512, 256, 256
T = 2048
ROWS = B * T

NC, NS = 2, 16
NW = NC * NS
RPW = ROWS // NW
CH = 64
NCH = RPW // CH
NBUF = 4
PRE = 2
VPC = CH // 16


def _ln_relu(y, scale, bias, ones_col):
    yb = y.astype(jnp.bfloat16)
    s1 = jnp.dot(yb, ones_col, preferred_element_type=jnp.float32)[:, 0:1]
    s2 = jnp.dot(yb * yb, ones_col, preferred_element_type=jnp.float32)[:, 0:1]
    mu = s1 * (1.0 / F)
    var = s2 * (1.0 / F) - mu * mu
    return jnp.maximum((y - mu) * lax.rsqrt(var + 1e-5) * scale + bias, 0.0)


def _conv3(h, w, bias):
    z = jnp.zeros((1, h.shape[1]), h.dtype)
    hprev = jnp.concatenate([z, h[:-1]], axis=0)
    hnext = jnp.concatenate([h[1:], z], axis=0)
    y = (jnp.dot(hprev, w[0], preferred_element_type=jnp.float32)
         + jnp.dot(h, w[1], preferred_element_type=jnp.float32)
         + jnp.dot(hnext, w[2], preferred_element_type=jnp.float32))
    return y + bias


PB = 4


def _pred_body(x_ref, w1_ref, b1_ref, s1_ref, g1_ref, w2_ref, b2_ref, s2_ref,
               g2_ref, lw_ref, lb_ref, out_ref):
    ones_col = jnp.ones((F, 128), jnp.bfloat16)
    for sb in range(PB):
        xb = x_ref[sb].astype(jnp.bfloat16)
        h = _ln_relu(_conv3(xb, w1_ref[...], b1_ref[...]), s1_ref[...],
                     g1_ref[...], ones_col)
        h = _ln_relu(_conv3(h.astype(jnp.bfloat16), w2_ref[...], b2_ref[...]),
                     s2_ref[...], g2_ref[...], ones_col)
        dpo = jnp.dot(h.astype(jnp.bfloat16), lw_ref[...],
                      preferred_element_type=jnp.float32)[:, 0:1]
        dpo = jnp.maximum(dpo + lb_ref[0, 0], 0.0)
        out_ref[sb] = dpo.reshape(1, L)


def _predictor(x, w1, b1, s1, g1, w2, b2, s2, g2, lw, lb):
    full = lambda a: pl.BlockSpec(a.shape, lambda b: (0,) * a.ndim)
    return pl.pallas_call(
        _pred_body,
        grid=(B // PB,),
        in_specs=[pl.BlockSpec((PB, L, D), lambda b: (b, 0, 0)),
                  full(w1), full(b1), full(s1), full(g1),
                  full(w2), full(b2), full(s2), full(g2),
                  full(lw), full(lb)],
        out_specs=pl.BlockSpec((PB, 1, L), lambda b: (b, 0, 0)),
        out_shape=jax.ShapeDtypeStruct((B, 1, L), jnp.float32),
    )(x, w1, b1, s1, g1, w2, b2, s2, g2, lw, lb)


def _sc_body(x_hbm, tgt_hbm, mel_hbm, out_hbm, tgt_v, cum_v, idx_v, mel_v,
             buf0, buf1, buf2, buf3, zbuf,
             gs0, gs1, gs2, gs3, ss0, ss1, ss2, ss3):
    wid = lax.axis_index("s") * NC + lax.axis_index("c")
    b = wid // 2
    half = wid % 2
    bufs = (buf0, buf1, buf2, buf3)
    gsems, ssems = (gs0, gs1, gs2, gs3), (ss0, ss1, ss2, ss3)

    def t_start(cnk):
        return (2 * cnk + half) * CH

    pltpu.sync_copy(tgt_hbm.at[pl.ds(b * L, L)], tgt_v)
    pltpu.sync_copy(mel_hbm, mel_v)
    mel_last = (mel_v[...][0] - 1).astype(jnp.float32)
    lane = lax.broadcasted_iota(jnp.int32, (16,), 0)

    def cs_body(i, carry):
        v = tgt_v[pl.ds(i * 16, 16)].astype(jnp.float32)
        cum_v[pl.ds(i * 16, 16)] = v
        for s in (1, 2, 4, 8):
            g = plsc.load_gather(cum_v, [jnp.maximum(lane - s, 0) + i * 16])
            v = v + jnp.where(lane >= s, g, 0.0)
            cum_v[pl.ds(i * 16, 16)] = v
        v = v + carry
        cum_v[pl.ds(i * 16, 16)] = v
        return v[15]

    total = lax.fori_loop(0, L // 16, cs_body, jnp.float32(0))

    def n_real(cnk):
        ts = jnp.float32(0) + t_start(cnk).astype(jnp.float32)
        return jnp.where(total > mel_last, jnp.float32(CH),
                         jnp.clip(total - ts, 0.0, jnp.float32(CH))
                         ).astype(jnp.int32)

    def search_chunk(cnk):
        def bs(j, carry):
            te = jnp.minimum(
                (t_start(cnk) + j * 16 + lane).astype(jnp.float32), mel_last)
            lo = jnp.zeros((16,), jnp.int32)
            for h in (256, 128, 64, 32, 16, 8, 4, 2, 1):
                cval = plsc.load_gather(cum_v, [lo + (h - 1)])
                lo = lo + jnp.where(cval <= te, h, 0)
            idx_v[pl.ds(cnk * CH + j * 16, 16)] = jnp.minimum(lo, L - 1) + b * L
            return carry
        lax.fori_loop(0, VPC, bs, 0)

    def gather(cnk, i):
        return pltpu.async_copy(
            x_hbm.at[idx_v.at[pl.ds(cnk * CH, CH)]], bufs[i], gsems[i])

    def out_slice(cnk):
        return out_hbm.at[pl.ds(b * T + t_start(cnk), CH)]

    def zero_tail(buf, zstart):
        def zrow(r, carry):
            for k in range(D // 16):
                buf[r, pl.ds(k * 16, 16)] = jnp.zeros((16,), jnp.float32)
            return carry
        lax.fori_loop(zstart, CH, zrow, 0)

    nr = [n_real(cnk) for cnk in range(NCH)]
    has_real = [nr[cnk] > 0 for cnk in range(NCH)]

    def wait_store(cnk, i):
        pltpu.make_async_copy(bufs[i], out_slice(cnk), ssems[i]).wait()

    for cnk in range(PRE):
        @pl.when(has_real[cnk])
        def _(cnk=cnk):
            search_chunk(cnk)
            gather(cnk, cnk % NBUF)
    zero_tail(zbuf, 0)

    store_waited = [False] * NCH
    for cnk in range(NCH):
        i = cnk % NBUF
        nxt = cnk + PRE
        if nxt < NCH:
            j = nxt % NBUF
            prev = nxt - NBUF
            if prev >= 0:
                wait_store(prev, j)
                store_waited[prev] = True

            @pl.when(has_real[nxt])
            def _(nxt=nxt, j=j):
                search_chunk(nxt)
                gather(nxt, j)

        @pl.when(has_real[cnk])
        def _(cnk=cnk, i=i):
            pltpu.make_async_copy(
                x_hbm.at[idx_v.at[pl.ds(cnk * CH, CH)]], bufs[i],
                gsems[i]).wait()
            zero_tail(bufs[i], nr[cnk])
            pltpu.async_copy(bufs[i], out_slice(cnk), ssems[i])

        @pl.when(jnp.logical_not(has_real[cnk]))
        def _(cnk=cnk, i=i):
            pltpu.async_copy(zbuf, out_slice(cnk), ssems[i])

    for cnk in range(NCH):
        if not store_waited[cnk]:
            wait_store(cnk, cnk % NBUF)


@functools.cache
def _make_sc_expand():
    return pl.kernel(
        _sc_body,
        mesh=plsc.VectorSubcoreMesh(core_axis_name="c", subcore_axis_name="s"),
        compiler_params=pltpu.CompilerParams(needs_layout_passes=False,
                                             disable_bounds_checks=True),
        out_type=jax.ShapeDtypeStruct((ROWS, D), jnp.float32),
        scratch_types=[
            pltpu.VMEM((L,), jnp.int32),
            pltpu.VMEM((L,), jnp.float32),
            pltpu.VMEM((RPW,), jnp.int32),
            pltpu.VMEM((16,), jnp.int32),
            pltpu.VMEM((CH, D), jnp.float32),
            pltpu.VMEM((CH, D), jnp.float32),
            pltpu.VMEM((CH, D), jnp.float32),
            pltpu.VMEM((CH, D), jnp.float32),
            pltpu.VMEM((CH, D), jnp.float32),
            pltpu.SemaphoreType.DMA,
            pltpu.SemaphoreType.DMA,
            pltpu.SemaphoreType.DMA,
            pltpu.SemaphoreType.DMA,
            pltpu.SemaphoreType.DMA,
            pltpu.SemaphoreType.DMA,
            pltpu.SemaphoreType.DMA,
            pltpu.SemaphoreType.DMA,
        ],
    )


def _sc_expand(x_flat, tgt_flat, mel16):
    return _make_sc_expand()(x_flat, tgt_flat, mel16)


def kernel(x, alpha, target, mel_max_length, conv1_w, conv1_b, ln1_scale,
           ln1_bias, conv2_w, conv2_b, ln2_scale, ln2_bias, lin_w, lin_b):
    del alpha
    mel16 = jnp.full((16,), mel_max_length, jnp.int32)
    output = _sc_expand(x.reshape(B * L, D), target.reshape(B * L),
                        mel16).reshape(B, T, D)

    w1 = jnp.transpose(conv1_w, (2, 1, 0)).astype(jnp.bfloat16)
    w2 = jnp.transpose(conv2_w, (2, 1, 0)).astype(jnp.bfloat16)
    lw = jnp.pad(lin_w, ((0, 0), (0, 127))).astype(jnp.bfloat16)
    dpo = _predictor(
        x, w1, conv1_b.reshape(1, F), ln1_scale.reshape(1, F),
        ln1_bias.reshape(1, F), w2, conv2_b.reshape(1, F),
        ln2_scale.reshape(1, F), ln2_bias.reshape(1, F),
        lw, lin_b.reshape(1, 1)).reshape(B, L)
    return output, dpo


# speedup vs baseline: 12.0329x; 1.0072x over previous
import functools

import jax
import jax.numpy as jnp
from jax import lax
from jax.experimental import pallas as pl
from jax.experimental.pallas import tpu as pltpu
from jax.experimental.pallas import tpu_sc as plsc

B, L, D, F = 16, 512, 256, 256
T = 2048
ROWS = B * T

NC, NS = 2, 16
NW = NC * NS
RPW = ROWS // NW
CH = 64
NCH = RPW // CH
NBUF = 6
PRE = 3
VPC = CH // 16


def _ln_relu(y, scale, bias, ones_col):
    yb = y.astype(jnp.bfloat16)
    s1 = jnp.dot(yb, ones_col, preferred_element_type=jnp.float32)[:, 0:1]
    s2 = jnp.dot(yb * yb, ones_col, preferred_element_type=jnp.float32)[:, 0:1]
    mu = s1 * (1.0 / F)
    var = s2 * (1.0 / F) - mu * mu
    return jnp.maximum((y - mu) * lax.rsqrt(var + 1e-5) * scale + bias, 0.0)


def _conv3(h, w, bias):
    z = jnp.zeros((1, h.shape[1]), h.dtype)
    hprev = jnp.concatenate([z, h[:-1]], axis=0)
    hnext = jnp.concatenate([h[1:], z], axis=0)
    y = (jnp.dot(hprev, w[0], preferred_element_type=jnp.float32)
         + jnp.dot(h, w[1], preferred_element_type=jnp.float32)
         + jnp.dot(hnext, w[2], preferred_element_type=jnp.float32))
    return y + bias


PB = 4


def _pred_body(x_ref, w1_ref, b1_ref, s1_ref, g1_ref, w2_ref, b2_ref, s2_ref,
               g2_ref, lw_ref, lb_ref, out_ref):
    ones_col = jnp.ones((F, 128), jnp.bfloat16)
    for sb in range(PB):
        xb = x_ref[sb].astype(jnp.bfloat16)
        h = _ln_relu(_conv3(xb, w1_ref[...], b1_ref[...]), s1_ref[...],
                     g1_ref[...], ones_col)
        h = _ln_relu(_conv3(h.astype(jnp.bfloat16), w2_ref[...], b2_ref[...]),
                     s2_ref[...], g2_ref[...], ones_col)
        dpo = jnp.dot(h.astype(jnp.bfloat16), lw_ref[...],
                      preferred_element_type=jnp.float32)[:, 0:1]
        dpo = jnp.maximum(dpo + lb_ref[0, 0], 0.0)
        out_ref[sb] = dpo.reshape(1, L)


def _predictor(x, w1, b1, s1, g1, w2, b2, s2, g2, lw, lb):
    full = lambda a: pl.BlockSpec(a.shape, lambda b: (0,) * a.ndim)
    return pl.pallas_call(
        _pred_body,
        grid=(B // PB,),
        in_specs=[pl.BlockSpec((PB, L, D), lambda b: (b, 0, 0)),
                  full(w1), full(b1), full(s1), full(g1),
                  full(w2), full(b2), full(s2), full(g2),
                  full(lw), full(lb)],
        out_specs=pl.BlockSpec((PB, 1, L), lambda b: (b, 0, 0)),
        out_shape=jax.ShapeDtypeStruct((B, 1, L), jnp.float32),
    )(x, w1, b1, s1, g1, w2, b2, s2, g2, lw, lb)


def _sc_body(x_hbm, tgt_hbm, mel_hbm, out_hbm, tgt_v, cum_v, idx_v, mel_v,
             buf0, buf1, buf2, buf3, buf4, buf5, zbuf,
             gs0, gs1, gs2, gs3, gs4, gs5, ss0, ss1, ss2, ss3, ss4, ss5):
    wid = lax.axis_index("s") * NC + lax.axis_index("c")
    b = wid // 2
    half = wid % 2
    bufs = (buf0, buf1, buf2, buf3, buf4, buf5)
    gsems = (gs0, gs1, gs2, gs3, gs4, gs5)
    ssems = (ss0, ss1, ss2, ss3, ss4, ss5)

    def t_start(cnk):
        return (2 * cnk + half) * CH

    pltpu.sync_copy(tgt_hbm.at[pl.ds(b * L, L)], tgt_v)
    pltpu.sync_copy(mel_hbm, mel_v)
    mel_last = (mel_v[...][0] - 1).astype(jnp.float32)
    lane = lax.broadcasted_iota(jnp.int32, (16,), 0)

    def cs_body(i, carry):
        v = tgt_v[pl.ds(i * 16, 16)].astype(jnp.float32)
        cum_v[pl.ds(i * 16, 16)] = v
        for s in (1, 2, 4, 8):
            g = plsc.load_gather(cum_v, [jnp.maximum(lane - s, 0) + i * 16])
            v = v + jnp.where(lane >= s, g, 0.0)
            cum_v[pl.ds(i * 16, 16)] = v
        v = v + carry
        cum_v[pl.ds(i * 16, 16)] = v
        return v[15]

    total = lax.fori_loop(0, L // 16, cs_body, jnp.float32(0))

    def n_real(cnk):
        ts = jnp.float32(0) + t_start(cnk).astype(jnp.float32)
        return jnp.where(total > mel_last, jnp.float32(CH),
                         jnp.clip(total - ts, 0.0, jnp.float32(CH))
                         ).astype(jnp.int32)

    def search_chunk(cnk):
        def bs(j, carry):
            te = jnp.minimum(
                (t_start(cnk) + j * 16 + lane).astype(jnp.float32), mel_last)
            lo = jnp.zeros((16,), jnp.int32)
            for h in (256, 128, 64, 32, 16, 8, 4, 2, 1):
                cval = plsc.load_gather(cum_v, [lo + (h - 1)])
                lo = lo + jnp.where(cval <= te, h, 0)
            idx_v[pl.ds(cnk * CH + j * 16, 16)] = jnp.minimum(lo, L - 1) + b * L
            return carry
        lax.fori_loop(0, VPC, bs, 0)

    def gather(cnk, i):
        return pltpu.async_copy(
            x_hbm.at[idx_v.at[pl.ds(cnk * CH, CH)]], bufs[i], gsems[i])

    def out_slice(cnk):
        return out_hbm.at[pl.ds(b * T + t_start(cnk), CH)]

    def zero_tail(buf, zstart):
        def zrow(r, carry):
            for k in range(D // 16):
                buf[r, pl.ds(k * 16, 16)] = jnp.zeros((16,), jnp.float32)
            return carry
        lax.fori_loop(zstart, CH, zrow, 0)

    nr = [n_real(cnk) for cnk in range(NCH)]
    has_real = [nr[cnk] > 0 for cnk in range(NCH)]

    def wait_store(cnk, i):
        pltpu.make_async_copy(bufs[i], out_slice(cnk), ssems[i]).wait()

    for cnk in range(PRE):
        @pl.when(has_real[cnk])
        def _(cnk=cnk):
            search_chunk(cnk)
            gather(cnk, cnk % NBUF)
    zero_tail(zbuf, 0)

    store_waited = [False] * NCH
    for cnk in range(NCH):
        i = cnk % NBUF
        nxt = cnk + PRE
        if nxt < NCH:
            j = nxt % NBUF
            prev = nxt - NBUF
            if prev >= 0:
                wait_store(prev, j)
                store_waited[prev] = True

            @pl.when(has_real[nxt])
            def _(nxt=nxt, j=j):
                search_chunk(nxt)
                gather(nxt, j)

        @pl.when(has_real[cnk])
        def _(cnk=cnk, i=i):
            pltpu.make_async_copy(
                x_hbm.at[idx_v.at[pl.ds(cnk * CH, CH)]], bufs[i],
                gsems[i]).wait()
            zero_tail(bufs[i], nr[cnk])
            pltpu.async_copy(bufs[i], out_slice(cnk), ssems[i])

        @pl.when(jnp.logical_not(has_real[cnk]))
        def _(cnk=cnk, i=i):
            pltpu.async_copy(zbuf, out_slice(cnk), ssems[i])

    for cnk in range(NCH):
        if not store_waited[cnk]:
            wait_store(cnk, cnk % NBUF)


@functools.cache
def _make_sc_expand():
    return pl.kernel(
        _sc_body,
        mesh=plsc.VectorSubcoreMesh(core_axis_name="c", subcore_axis_name="s"),
        compiler_params=pltpu.CompilerParams(needs_layout_passes=False,
                                             disable_bounds_checks=True),
        out_type=jax.ShapeDtypeStruct((ROWS, D), jnp.float32),
        scratch_types=[
            pltpu.VMEM((L,), jnp.int32),
            pltpu.VMEM((L,), jnp.float32),
            pltpu.VMEM((RPW,), jnp.int32),
            pltpu.VMEM((16,), jnp.int32),
        ] + [pltpu.VMEM((CH, D), jnp.float32)] * (NBUF + 1)
          + [pltpu.SemaphoreType.DMA] * (2 * NBUF),
    )


def _sc_expand(x_flat, tgt_flat, mel16):
    return _make_sc_expand()(x_flat, tgt_flat, mel16)


def kernel(x, alpha, target, mel_max_length, conv1_w, conv1_b, ln1_scale,
           ln1_bias, conv2_w, conv2_b, ln2_scale, ln2_bias, lin_w, lin_b):
    del alpha
    mel16 = jnp.full((16,), mel_max_length, jnp.int32)
    output = _sc_expand(x.reshape(B * L, D), target.reshape(B * L),
                        mel16).reshape(B, T, D)

    w1 = jnp.transpose(conv1_w, (2, 1, 0)).astype(jnp.bfloat16)
    w2 = jnp.transpose(conv2_w, (2, 1, 0)).astype(jnp.bfloat16)
    lw = jnp.pad(lin_w, ((0, 0), (0, 127))).astype(jnp.bfloat16)
    dpo = _predictor(
        x, w1, conv1_b.reshape(1, F), ln1_scale.reshape(1, F),
        ln1_bias.reshape(1, F), w2, conv2_b.reshape(1, F),
        ln2_scale.reshape(1, F), ln2_bias.reshape(1, F),
        lw, lin_b.reshape(1, 1)).reshape(B, L)
    return output, dpo


# speedup vs baseline: 12.1884x; 1.0129x over previous
import functools

import jax
import jax.numpy as jnp
from jax import lax
from jax.experimental import pallas as pl
from jax.experimental.pallas import tpu as pltpu
from jax.experimental.pallas import tpu_sc as plsc

B, L, D, F = 16, 512, 256, 256
T = 2048
ROWS = B * T

NC, NS = 2, 16
NW = NC * NS
RPW = ROWS // NW
CH = 64
NCH = RPW // CH
NBUF = 6
PRE = 4
VPC = CH // 16


def _ln_relu(y, scale, bias, ones_col):
    yb = y.astype(jnp.bfloat16)
    s1 = jnp.dot(yb, ones_col, preferred_element_type=jnp.float32)[:, 0:1]
    s2 = jnp.dot(yb * yb, ones_col, preferred_element_type=jnp.float32)[:, 0:1]
    mu = s1 * (1.0 / F)
    var = s2 * (1.0 / F) - mu * mu
    return jnp.maximum((y - mu) * lax.rsqrt(var + 1e-5) * scale + bias, 0.0)


def _conv3(h, w, bias):
    z = jnp.zeros((1, h.shape[1]), h.dtype)
    hprev = jnp.concatenate([z, h[:-1]], axis=0)
    hnext = jnp.concatenate([h[1:], z], axis=0)
    y = (jnp.dot(hprev, w[0], preferred_element_type=jnp.float32)
         + jnp.dot(h, w[1], preferred_element_type=jnp.float32)
         + jnp.dot(hnext, w[2], preferred_element_type=jnp.float32))
    return y + bias


PB = 4


def _pred_body(x_ref, w1_ref, b1_ref, s1_ref, g1_ref, w2_ref, b2_ref, s2_ref,
               g2_ref, lw_ref, lb_ref, out_ref):
    ones_col = jnp.ones((F, 128), jnp.bfloat16)
    for sb in range(PB):
        xb = x_ref[sb].astype(jnp.bfloat16)
        h = _ln_relu(_conv3(xb, w1_ref[...], b1_ref[...]), s1_ref[...],
                     g1_ref[...], ones_col)
        h = _ln_relu(_conv3(h.astype(jnp.bfloat16), w2_ref[...], b2_ref[...]),
                     s2_ref[...], g2_ref[...], ones_col)
        dpo = jnp.dot(h.astype(jnp.bfloat16), lw_ref[...],
                      preferred_element_type=jnp.float32)[:, 0:1]
        dpo = jnp.maximum(dpo + lb_ref[0, 0], 0.0)
        out_ref[sb] = dpo.reshape(1, L)


def _predictor(x, w1, b1, s1, g1, w2, b2, s2, g2, lw, lb):
    full = lambda a: pl.BlockSpec(a.shape, lambda b: (0,) * a.ndim)
    return pl.pallas_call(
        _pred_body,
        grid=(B // PB,),
        in_specs=[pl.BlockSpec((PB, L, D), lambda b: (b, 0, 0)),
                  full(w1), full(b1), full(s1), full(g1),
                  full(w2), full(b2), full(s2), full(g2),
                  full(lw), full(lb)],
        out_specs=pl.BlockSpec((PB, 1, L), lambda b: (b, 0, 0)),
        out_shape=jax.ShapeDtypeStruct((B, 1, L), jnp.float32),
    )(x, w1, b1, s1, g1, w2, b2, s2, g2, lw, lb)


def _sc_body(x_hbm, tgt_hbm, mel_hbm, out_hbm, tgt_v, cum_v, idx_v, mel_v,
             buf0, buf1, buf2, buf3, buf4, buf5, zbuf,
             gs0, gs1, gs2, gs3, gs4, gs5, ss0, ss1, ss2, ss3, ss4, ss5):
    wid = lax.axis_index("s") * NC + lax.axis_index("c")
    b = wid // 2
    half = wid % 2
    bufs = (buf0, buf1, buf2, buf3, buf4, buf5)
    gsems = (gs0, gs1, gs2, gs3, gs4, gs5)
    ssems = (ss0, ss1, ss2, ss3, ss4, ss5)

    def t_start(cnk):
        return (2 * cnk + half) * CH

    pltpu.sync_copy(tgt_hbm.at[b], tgt_v)
    pltpu.sync_copy(mel_hbm, mel_v)
    mel_last = (mel_v[...][0] - 1).astype(jnp.float32)
    lane = lax.broadcasted_iota(jnp.int32, (16,), 0)

    def cs_body(i, carry):
        v = tgt_v[pl.ds(i * 16, 16)].astype(jnp.float32)
        cum_v[pl.ds(i * 16, 16)] = v
        for s in (1, 2, 4, 8):
            g = plsc.load_gather(cum_v, [jnp.maximum(lane - s, 0) + i * 16])
            v = v + jnp.where(lane >= s, g, 0.0)
            cum_v[pl.ds(i * 16, 16)] = v
        v = v + carry
        cum_v[pl.ds(i * 16, 16)] = v
        return v[15]

    total = lax.fori_loop(0, L // 16, cs_body, jnp.float32(0))

    def n_real(cnk):
        ts = jnp.float32(0) + t_start(cnk).astype(jnp.float32)
        return jnp.where(total > mel_last, jnp.float32(CH),
                         jnp.clip(total - ts, 0.0, jnp.float32(CH))
                         ).astype(jnp.int32)

    def search_chunk(cnk):
        def bs(j, carry):
            te = jnp.minimum(
                (t_start(cnk) + j * 16 + lane).astype(jnp.float32), mel_last)
            lo = jnp.zeros((16,), jnp.int32)
            for h in (256, 128, 64, 32, 16, 8, 4, 2, 1):
                cval = plsc.load_gather(cum_v, [lo + (h - 1)])
                lo = lo + jnp.where(cval <= te, h, 0)
            idx_v[pl.ds(cnk * CH + j * 16, 16)] = jnp.minimum(lo, L - 1) + b * L
            return carry
        lax.fori_loop(0, VPC, bs, 0)

    def gather(cnk, i):
        return pltpu.async_copy(
            x_hbm.at[idx_v.at[pl.ds(cnk * CH, CH)]], bufs[i], gsems[i])

    def out_slice(cnk):
        return out_hbm.at[pl.ds(b * T + t_start(cnk), CH)]

    def zero_tail(buf, zstart):
        def zrow(r, carry):
            for k in range(D // 16):
                buf[r, pl.ds(k * 16, 16)] = jnp.zeros((16,), jnp.float32)
            return carry
        lax.fori_loop(zstart, CH, zrow, 0)

    nr = [n_real(cnk) for cnk in range(NCH)]
    has_real = [nr[cnk] > 0 for cnk in range(NCH)]

    def wait_store(cnk, i):
        pltpu.make_async_copy(bufs[i], out_slice(cnk), ssems[i]).wait()

    for cnk in range(PRE):
        @pl.when(has_real[cnk])
        def _(cnk=cnk):
            search_chunk(cnk)
            gather(cnk, cnk % NBUF)
    zero_tail(zbuf, 0)

    store_waited = [False] * NCH
    for cnk in range(NCH):
        i = cnk % NBUF
        nxt = cnk + PRE
        if nxt < NCH:
            j = nxt % NBUF
            prev = nxt - NBUF
            if prev >= 0:
                wait_store(prev, j)
                store_waited[prev] = True

            @pl.when(has_real[nxt])
            def _(nxt=nxt, j=j):
                search_chunk(nxt)
                gather(nxt, j)

        @pl.when(has_real[cnk])
        def _(cnk=cnk, i=i):
            pltpu.make_async_copy(
                x_hbm.at[idx_v.at[pl.ds(cnk * CH, CH)]], bufs[i],
                gsems[i]).wait()
            zero_tail(bufs[i], nr[cnk])
            pltpu.async_copy(bufs[i], out_slice(cnk), ssems[i])

        @pl.when(jnp.logical_not(has_real[cnk]))
        def _(cnk=cnk, i=i):
            pltpu.async_copy(zbuf, out_slice(cnk), ssems[i])

    for cnk in range(NCH):
        if not store_waited[cnk]:
            wait_store(cnk, cnk % NBUF)


@functools.cache
def _make_sc_expand():
    return pl.kernel(
        _sc_body,
        mesh=plsc.VectorSubcoreMesh(core_axis_name="c", subcore_axis_name="s"),
        compiler_params=pltpu.CompilerParams(needs_layout_passes=False,
                                             disable_bounds_checks=True),
        out_type=jax.ShapeDtypeStruct((ROWS, D), jnp.float32),
        scratch_types=[
            pltpu.VMEM((L,), jnp.int32),
            pltpu.VMEM((L,), jnp.float32),
            pltpu.VMEM((RPW,), jnp.int32),
            pltpu.VMEM((16,), jnp.int32),
        ] + [pltpu.VMEM((CH, D), jnp.float32)] * (NBUF + 1)
          + [pltpu.SemaphoreType.DMA] * (2 * NBUF),
    )


def _sc_expand(x_flat, tgt_flat, mel16):
    return _make_sc_expand()(x_flat, tgt_flat, mel16)


def kernel(x, alpha, target, mel_max_length, conv1_w, conv1_b, ln1_scale,
           ln1_bias, conv2_w, conv2_b, ln2_scale, ln2_bias, lin_w, lin_b):
    del alpha
    mel16 = jnp.full((16,), mel_max_length, jnp.int32)
    output = _sc_expand(x.reshape(B * L, D), target,
                        mel16).reshape(B, T, D)

    w1 = jnp.transpose(conv1_w, (2, 1, 0)).astype(jnp.bfloat16)
    w2 = jnp.transpose(conv2_w, (2, 1, 0)).astype(jnp.bfloat16)
    lw = jnp.pad(lin_w, ((0, 0), (0, 127))).astype(jnp.bfloat16)
    dpo = _predictor(
        x, w1, conv1_b.reshape(1, F), ln1_scale.reshape(1, F),
        ln1_bias.reshape(1, F), w2, conv2_b.reshape(1, F),
        ln2_scale.reshape(1, F), ln2_bias.reshape(1, F),
        lw, lin_b.reshape(1, 1)).reshape(B, L)
    return output, dpo
